# Initial kernel scaffold; baseline (speedup 1.0000x reference)
import jax
import jax.numpy as jnp
from jax.experimental import pallas as pl


def kernel(item, cat, W_item, W_cat):
    raise NotImplementedError("write your pallas kernel here")


# speedup vs baseline: 2.2730x; 2.2730x over previous
import functools

import jax
import jax.numpy as jnp
from jax import lax
from jax.experimental import pallas as pl
from jax.experimental.pallas import tpu as pltpu
from jax.experimental.pallas import tpu_sc as plsc

_B = 4096
_L = 200
_V_ITEM = 100000
_D_ITEM = 64
_V_CAT = 1000
_D_CAT = 32
_D = _D_ITEM + _D_CAT

_NC = 2
_NS = 16
_NW = _NC * _NS

_NB = 16
_CHUNKS = _NB * _L // 16
_NBLK = _B // _NB


@functools.partial(
    pl.kernel,
    mesh=plsc.VectorSubcoreMesh(core_axis_name="c", subcore_axis_name="s"),
    compiler_params=pltpu.CompilerParams(
        needs_layout_passes=False, use_tc_tiling_on_sc=False),
    out_type=jax.ShapeDtypeStruct((_B, _D, _L), jnp.float32),
    scratch_types=[
        pltpu.VMEM((_V_ITEM,), jnp.float32),
        pltpu.VMEM((_NB * _L,), jnp.int32),
        pltpu.VMEM((_NB * _L,), jnp.float32),
        pltpu.SemaphoreType.DMA,
    ],
)
def _seq_embed_sc(wit_hbm, wct_hbm, item_hbm, cat_hbm, out_hbm,
                  tab_v, idx_v, dat_v, sem):
    wid = lax.axis_index("s") * _NC + lax.axis_index("c")

    def run_channel(ch, idx_src_hbm):
        def blk_body(blk, carry):
            b0 = blk * _NB
            pltpu.sync_copy(idx_src_hbm.at[pl.ds(b0 * _L, _NB * _L)], idx_v)

            def chunk(i, c):
                ii = i * 16
                idx = idx_v[pl.ds(ii, 16)]
                dat_v[pl.ds(ii, 16)] = plsc.load_gather(tab_v, [idx])
                return c

            lax.fori_loop(0, _CHUNKS, chunk, 0)
            handles = [
                pltpu.async_copy(dat_v.at[pl.ds(r * _L, _L)],
                                 out_hbm.at[b0 + r, ch, :], sem)
                for r in range(_NB)
            ]
            for h in handles:
                h.wait()
            return carry

        lax.fori_loop(0, _NBLK, blk_body, 0)

    pltpu.sync_copy(wit_hbm.at[wid], tab_v)
    run_channel(wid, item_hbm)
    pltpu.sync_copy(wit_hbm.at[wid + _NW], tab_v)
    run_channel(wid + _NW, item_hbm)
    pltpu.sync_copy(wct_hbm.at[wid], tab_v.at[pl.ds(0, _V_CAT)])
    run_channel(wid + 2 * _NW, cat_hbm)


def kernel(item, cat, W_item, W_cat):
    wit = W_item.T
    wct = W_cat.T
    item_flat = item.reshape(-1).astype(jnp.int32)
    cat_flat = cat.reshape(-1).astype(jnp.int32)
    return _seq_embed_sc(wit, wct, item_flat, cat_flat)


# speedup vs baseline: 4.0236x; 1.7702x over previous
import functools

import jax
import jax.numpy as jnp
from jax import lax
from jax.experimental import pallas as pl
from jax.experimental.pallas import tpu as pltpu
from jax.experimental.pallas import tpu_sc as plsc

_B = 4096
_L = 200
_V_ITEM = 100000
_D_ITEM = 64
_V_CAT = 1000
_D_CAT = 32
_D = _D_ITEM + _D_CAT

_NC = 2
_NS = 16
_NW = _NC * _NS

_NB = 16
_BLK = _NB * _L
_NBLK = _B // _NB


@functools.partial(
    pl.kernel,
    mesh=plsc.VectorSubcoreMesh(core_axis_name="c", subcore_axis_name="s"),
    compiler_params=pltpu.CompilerParams(
        needs_layout_passes=False, use_tc_tiling_on_sc=False),
    out_type=jax.ShapeDtypeStruct((_NBLK, _NB, _D, _L), jnp.float32),
    scratch_types=[
        pltpu.VMEM((_V_ITEM,), jnp.float32),
        pltpu.VMEM((_BLK,), jnp.int32),
        pltpu.VMEM((_BLK,), jnp.int32),
        pltpu.VMEM((_NB, _L), jnp.float32),
        pltpu.VMEM((_NB, _L), jnp.float32),
        pltpu.SemaphoreType.DMA,
        pltpu.SemaphoreType.DMA,
        pltpu.SemaphoreType.DMA,
        pltpu.SemaphoreType.DMA,
    ],
)
def _seq_embed_sc(wit_hbm, wct_hbm, item_hbm, cat_hbm, out_hbm,
                  tab_v, idx_v0, idx_v1, dat_v0, dat_v1,
                  sem_i0, sem_i1, sem_o0, sem_o1):
    wid = lax.axis_index("s") * _NC + lax.axis_index("c")
    idx_vs = (idx_v0, idx_v1)
    dat_vs = (dat_v0, dat_v1)
    sem_is = (sem_i0, sem_i1)
    sem_os = (sem_o0, sem_o1)

    def run_channel(ch, idx_src_hbm):
        for ph in range(2):
            pltpu.async_copy(idx_src_hbm.at[pl.ds(ph * _BLK, _BLK)],
                             idx_vs[ph], sem_is[ph])

        def gather_block(idx_v, dat_v):
            @plsc.parallel_loop(0, _BLK, 16, unroll=8)
            def _chunk(ii):
                idx = idx_v[pl.ds(ii, 16)]
                vals = plsc.load_gather(tab_v, [idx])
                p = lax.iota(jnp.int32, 16) + ii
                row = p // _L
                col = p - row * _L
                plsc.store_scatter(dat_v, [row, col], vals)

        def pair_body(pr, carry):
            for ph in range(2):
                blk = 2 * pr + ph
                idx_v, dat_v = idx_vs[ph], dat_vs[ph]
                sem_i, sem_o = sem_is[ph], sem_os[ph]
                pltpu.make_async_copy(
                    idx_src_hbm.at[pl.ds(blk * _BLK, _BLK)], idx_v,
                    sem_i).wait()
                @pl.when(blk >= 2)
                def _():
                    pltpu.make_async_copy(
                        dat_v, out_hbm.at[blk - 2, :, ch, :], sem_o).wait()
                gather_block(idx_v, dat_v)
                @pl.when(blk + 2 < _NBLK)
                def _():
                    pltpu.async_copy(
                        idx_src_hbm.at[pl.ds((blk + 2) * _BLK, _BLK)],
                        idx_v, sem_i)
                pltpu.async_copy(dat_v, out_hbm.at[blk, :, ch, :], sem_o)
            return carry

        lax.fori_loop(0, _NBLK // 2, pair_body, 0)
        for ph in range(2):
            pltpu.make_async_copy(
                dat_vs[ph], out_hbm.at[_NBLK - 2 + ph, :, ch, :],
                sem_os[ph]).wait()

    pltpu.sync_copy(wit_hbm.at[wid], tab_v)
    run_channel(wid, item_hbm)
    pltpu.sync_copy(wit_hbm.at[wid + _NW], tab_v)
    run_channel(wid + _NW, item_hbm)
    pltpu.sync_copy(wct_hbm.at[wid], tab_v.at[pl.ds(0, _V_CAT)])
    run_channel(wid + 2 * _NW, cat_hbm)


def kernel(item, cat, W_item, W_cat):
    wit = W_item.T
    wct = W_cat.T
    item_flat = item.reshape(-1).astype(jnp.int32)
    cat_flat = cat.reshape(-1).astype(jnp.int32)
    out = _seq_embed_sc(wit, wct, item_flat, cat_flat)
    return out.reshape(_B, _D, _L)
